# Initial kernel scaffold; baseline (speedup 1.0000x reference)
import jax
import jax.numpy as jnp
from jax.experimental import pallas as pl


def kernel(adj1, feat1, W1, b1, W2, b2, Wg1, bg1, Wg2, bg2, Wp1, bp1, Wp2, bp2):
    raise NotImplementedError("write your pallas kernel here")


# speedup vs baseline: 1.2089x; 1.2089x over previous
import functools

import jax
import jax.numpy as jnp
from jax.experimental import pallas as pl
from jax.experimental.pallas import tpu as pltpu

TEMP = 0.5


def _head_kernel(feat_ref, W1_ref, b1_ref, W2_ref, b2_ref, Wg1_ref, out_ref):
    f1 = jnp.maximum(
        jnp.dot(feat_ref[...], W1_ref[...], preferred_element_type=jnp.float32)
        + b1_ref[...], 0.0)
    f2 = jnp.dot(f1, W2_ref[...], preferred_element_type=jnp.float32) + b2_ref[...]
    out_ref[...] = jnp.dot(f2, Wg1_ref[...], preferred_element_type=jnp.float32)


def _adj_mm1_kernel(adj_ref, sup_ref, Wg2_ref, bg1_ref, out_ref):
    acc = jnp.dot(adj_ref[...], sup_ref[...],
                  preferred_element_type=jnp.float32)
    h = jnp.maximum(acc + bg1_ref[...], 0.0)
    out_ref[...] = jnp.dot(h, Wg2_ref[...], preferred_element_type=jnp.float32)


def _adj_mm2_kernel(adj_ref, sup_ref, bg2_ref, Wp1_ref, bp1_ref, Wp2_ref,
                    bp2_ref, out_ref):
    acc = jnp.dot(adj_ref[...], sup_ref[...],
                  preferred_element_type=jnp.float32)
    logits = acc + bg2_ref[...]
    t = jnp.dot(logits, Wp1_ref[...],
                preferred_element_type=jnp.float32) + bp1_ref[...]
    t = jnp.where(t > 0, t, jnp.exp(jnp.minimum(t, 0.0)) - 1.0)
    z1 = jnp.dot(t, Wp2_ref[...],
                 preferred_element_type=jnp.float32) + bp2_ref[...]
    norm = jnp.sqrt(jnp.sum(z1 * z1, axis=1, keepdims=True))
    out_ref[...] = z1 / jnp.maximum(norm, 1e-12)


def _loss_kernel(znrow_ref, znall_ref, out_ref, rows_ref, *, bn, nj, n,
                 inv_temp):
    i = pl.program_id(0)
    j = pl.program_id(1)
    s = jax.lax.dot_general(
        znrow_ref[...], znall_ref[pl.ds(j * bn, bn), :],
        (((1,), (1,)), ((), ())),
        preferred_element_type=jnp.float32) * inv_temp
    rs = jnp.sum(jnp.exp(s), axis=1, keepdims=True)

    @pl.when(j == 0)
    def _():
        rows_ref[...] = rs

    @pl.when(j > 0)
    def _():
        rows_ref[...] += rs

    @pl.when(jnp.logical_and(i == 0, j == 0))
    def _():
        out_ref[...] = jnp.zeros((1, 1), jnp.float32)

    @pl.when(j == nj - 1)
    def _():
        zr = znrow_ref[...]
        darg = jnp.sum(zr * zr, axis=1, keepdims=True) * inv_temp
        x1 = rows_ref[...] + jnp.exp(darg)
        blk = jnp.sum(jnp.log(x1) - darg) * (1.0 / n)
        out_ref[...] += jnp.full((1, 1), blk, jnp.float32)


def kernel(adj1, feat1, W1, b1, W2, b2, Wg1, bg1, Wg2, bg2, Wp1, bp1, Wp2,
           bp2):
    n = adj1.shape[0]
    in_dim = feat1.shape[1]
    hid = Wg1.shape[1]
    out_dim = Wp1.shape[1]

    b1r = b1.reshape(1, -1)
    b2r = b2.reshape(1, -1)
    bg1r = bg1.reshape(1, -1)
    bg2r = bg2.reshape(1, -1)
    bp1r = bp1.reshape(1, -1)
    bp2r = bp2.reshape(1, -1)

    bm_head = n // 5
    sup1 = pl.pallas_call(
        _head_kernel,
        grid=(5,),
        in_specs=[
            pl.BlockSpec((bm_head, in_dim), lambda i: (i, 0)),
            pl.BlockSpec((in_dim, 64), lambda i: (0, 0)),
            pl.BlockSpec((1, 64), lambda i: (0, 0)),
            pl.BlockSpec((64, 32), lambda i: (0, 0)),
            pl.BlockSpec((1, 32), lambda i: (0, 0)),
            pl.BlockSpec((32, hid), lambda i: (0, 0)),
        ],
        out_specs=pl.BlockSpec((bm_head, hid), lambda i: (i, 0)),
        out_shape=jax.ShapeDtypeStruct((n, hid), jnp.float32),
    )(feat1, W1, b1r, W2, b2r, Wg1)

    bm = n // 25
    mm_grid = (n // bm,)
    adj_specs = [
        pl.BlockSpec((bm, n), lambda i: (i, 0)),
        pl.BlockSpec((n, hid), lambda i: (0, 0)),
    ]
    mm_params = pltpu.CompilerParams(dimension_semantics=("arbitrary",))

    sup2 = pl.pallas_call(
        _adj_mm1_kernel,
        grid=mm_grid,
        in_specs=adj_specs + [
            pl.BlockSpec((hid, hid), lambda i: (0, 0)),
            pl.BlockSpec((1, hid), lambda i: (0, 0)),
        ],
        out_specs=pl.BlockSpec((bm, hid), lambda i: (i, 0)),
        out_shape=jax.ShapeDtypeStruct((n, hid), jnp.float32),
        compiler_params=mm_params,
    )(adj1, sup1, Wg2, bg1r)

    zn = pl.pallas_call(
        _adj_mm2_kernel,
        grid=mm_grid,
        in_specs=adj_specs + [
            pl.BlockSpec((1, hid), lambda i: (0, 0)),
            pl.BlockSpec((hid, out_dim), lambda i: (0, 0)),
            pl.BlockSpec((1, out_dim), lambda i: (0, 0)),
            pl.BlockSpec((out_dim, hid), lambda i: (0, 0)),
            pl.BlockSpec((1, hid), lambda i: (0, 0)),
        ],
        out_specs=pl.BlockSpec((bm, hid), lambda i: (i, 0)),
        out_shape=jax.ShapeDtypeStruct((n, hid), jnp.float32),
        compiler_params=mm_params,
    )(adj1, sup2, bg2r, Wp1, bp1r, Wp2, bp2r)

    bm2, bn = n // 10, n // 5
    nj = n // bn
    total = pl.pallas_call(
        functools.partial(_loss_kernel, bn=bn, nj=nj, n=n,
                          inv_temp=1.0 / TEMP),
        grid=(n // bm2, nj),
        in_specs=[
            pl.BlockSpec((bm2, hid), lambda i, j: (i, 0)),
            pl.BlockSpec((n, hid), lambda i, j: (0, 0)),
        ],
        out_specs=pl.BlockSpec((1, 1), lambda i, j: (0, 0)),
        out_shape=jax.ShapeDtypeStruct((1, 1), jnp.float32),
        scratch_shapes=[pltpu.VMEM((bm2, 1), jnp.float32)],
        compiler_params=pltpu.CompilerParams(
            dimension_semantics=("arbitrary", "arbitrary")),
    )(zn, zn)

    return total[0, 0]
